# Initial kernel scaffold; baseline (speedup 1.0000x reference)
import jax
import jax.numpy as jnp
from jax.experimental import pallas as pl


def kernel(hoch_features, inp, params):
    raise NotImplementedError("write your pallas kernel here")


# speedup vs baseline: 1.8159x; 1.8159x over previous
import functools

import jax
import jax.numpy as jnp
import numpy as np
from jax import lax
from jax.experimental import pallas as pl
from jax.experimental.pallas import tpu as pltpu
from jax.experimental.pallas import tpu_sc as plsc

B, N, C = 8, 2048, 256
H, DH = 4, 64
S = 16
KNB = 32
KPAD = 48
R2 = 0.3 * 0.3


def _dotb(a, b):
    return jnp.dot(a.astype(jnp.bfloat16), b.astype(jnp.bfloat16),
                   preferred_element_type=jnp.float32)


def _attn_body(x_ref, wqkv_ref, bqkv_ref, wo_ref, bo_ref, out_ref):
    x = x_ref[0]
    qkv = _dotb(x, wqkv_ref[...]) + bqkv_ref[...]
    scale = 1.0 / np.sqrt(DH)
    o_cols = []
    for h in range(H):
        q = qkv[:, h * DH:(h + 1) * DH]
        k = qkv[:, C + h * DH:C + (h + 1) * DH]
        v = qkv[:, 2 * C + h * DH:2 * C + (h + 1) * DH]
        logits = lax.dot_general(q.astype(jnp.bfloat16), k.astype(jnp.bfloat16),
                                 (((1,), (1,)), ((), ())),
                                 preferred_element_type=jnp.float32) * scale
        m = jnp.max(logits, axis=-1, keepdims=True)
        p = jnp.exp(logits - m)
        p = p / jnp.sum(p, axis=-1, keepdims=True)
        o_cols.append(_dotb(p, v))
    o = jnp.concatenate(o_cols, axis=-1)
    out_ref[0] = x + _dotb(o, wo_ref[...]) + bo_ref[...]


def _attn_layer(x, wqkv, bqkv, wo, bo):
    return pl.pallas_call(
        _attn_body,
        grid=(B,),
        in_specs=[
            pl.BlockSpec((1, N, C), lambda b: (b, 0, 0)),
            pl.BlockSpec((C, 3 * C), lambda b: (0, 0)),
            pl.BlockSpec((1, 3 * C), lambda b: (0, 0)),
            pl.BlockSpec((C, C), lambda b: (0, 0)),
            pl.BlockSpec((1, C), lambda b: (0, 0)),
        ],
        out_specs=pl.BlockSpec((1, N, C), lambda b: (b, 0, 0)),
        out_shape=jax.ShapeDtypeStruct((B, N, C), jnp.float32),
    )(x, wqkv, bqkv, wo, bo)


def _bn_relu(h, g, be):
    m = jnp.mean(h, axis=0, keepdims=True)
    v = jnp.mean((h - m) ** 2, axis=0, keepdims=True)
    h = (h - m) / jnp.sqrt(v + 1e-5) * g + be
    return jnp.maximum(h, 0.0)


def _fg_body(x_ref, w0, b0, g0, e0, w1, b1, g1, e1, w2, b2, g2, e2, out_ref):
    h = _dotb(x_ref[...], w0[...]) + b0[...]
    h = _bn_relu(h, g0[...], e0[...])
    h = _dotb(h, w1[...]) + b1[...]
    h = _bn_relu(h, g1[...], e1[...])
    h = _dotb(h, w2[...]) + b2[...]
    h = _bn_relu(h, g2[...], e2[...])
    hr = h.reshape(B, N, S)
    mx = jnp.max(hr, axis=1, keepdims=True)
    ii = lax.broadcasted_iota(jnp.int32, (B, N, S), 1)
    out_ref[...] = jnp.min(jnp.where(hr == mx, ii, N), axis=1)


def _fg_argmax(xf, p):
    args = []
    for j, (ci, co) in enumerate(((C, 64), (64, 32), (32, S))):
        args += [p["fg_W%d" % j].T,
                 p["fg_b%d" % j].reshape(1, co),
                 p["fg_g%d" % j].reshape(1, co),
                 p["fg_be%d" % j].reshape(1, co)]
    return pl.pallas_call(
        _fg_body,
        out_shape=jax.ShapeDtypeStruct((B, S), jnp.int32),
    )(xf, *args)


def _ballquery_body(inp_hbm, idx_hbm, out_hbm, xv, yv, zv, idxrow, idxbuf,
                    outbuf):
    info = plsc.get_sparse_core_info()
    nc = info.num_cores
    wid = lax.axis_index("s") * nc + lax.axis_index("c")
    b = wid // 4
    soff = (wid % 4) * 4
    pltpu.sync_copy(inp_hbm.at[pl.ds((b * 3 + 0) * N, N)], xv)
    pltpu.sync_copy(inp_hbm.at[pl.ds((b * 3 + 1) * N, N)], yv)
    pltpu.sync_copy(inp_hbm.at[pl.ds((b * 3 + 2) * N, N)], zv)
    pltpu.sync_copy(idx_hbm.at[pl.ds(b * S, S)], idxrow.at[pl.ds(0, S)])
    lane = lax.iota(jnp.int32, 16)
    zero16 = jnp.zeros((16,), jnp.int32)
    for t in range(4):
        s_loc = soff + t
        qiv = plsc.load_gather(idxrow, [jnp.full((16,), s_loc, jnp.int32)])
        qx = plsc.load_gather(xv, [qiv])
        qy = plsc.load_gather(yv, [qiv])
        qz = plsc.load_gather(zv, [qiv])

        def body(ci, cnt):
            px = xv[pl.ds(ci * 16, 16)]
            py = yv[pl.ds(ci * 16, 16)]
            pz = zv[pl.ds(ci * 16, 16)]
            dx, dy, dz = px - qx, py - qy, pz - qz
            d2 = dx * dx + dy * dy + dz * dz
            msk = d2 <= R2
            vals = lane + ci * 16
            plsc.store_compressed(idxbuf.at[pl.ds(cnt, 16)], vals, mask=msk)
            return cnt + jnp.max(plsc.all_reduce_population_count(msk))

        cnt = lax.fori_loop(0, N // 16, body, jnp.int32(0))
        firstv = plsc.load_gather(idxbuf, [zero16])
        cntv = jnp.full((16,), cnt, jnp.int32)
        sel0 = jnp.where(lane < cntv, idxbuf[pl.ds(0, 16)], firstv)
        sel1 = jnp.where(lane + 16 < cntv, idxbuf[pl.ds(16, 16)], firstv)
        outbuf[pl.ds(0, 16)] = plsc.load_gather(xv, [sel0])
        outbuf[pl.ds(16, 16)] = plsc.load_gather(xv, [sel1])
        outbuf[pl.ds(32, 16)] = qx
        outbuf[pl.ds(48, 16)] = plsc.load_gather(yv, [sel0])
        outbuf[pl.ds(64, 16)] = plsc.load_gather(yv, [sel1])
        outbuf[pl.ds(80, 16)] = qy
        outbuf[pl.ds(96, 16)] = plsc.load_gather(zv, [sel0])
        outbuf[pl.ds(112, 16)] = plsc.load_gather(zv, [sel1])
        outbuf[pl.ds(128, 16)] = qz
        pltpu.sync_copy(outbuf,
                        out_hbm.at[pl.ds((b * S + s_loc) * (3 * KPAD), 3 * KPAD)])


def _ballquery(inp, idx):
    mesh = plsc.VectorSubcoreMesh(core_axis_name="c", subcore_axis_name="s")
    fn = functools.partial(
        pl.kernel,
        mesh=mesh,
        compiler_params=pltpu.CompilerParams(needs_layout_passes=False),
        out_type=jax.ShapeDtypeStruct((B * S * 3 * KPAD,), jnp.float32),
        scratch_types=[
            pltpu.VMEM((N,), jnp.float32),
            pltpu.VMEM((N,), jnp.float32),
            pltpu.VMEM((N,), jnp.float32),
            pltpu.VMEM((128,), jnp.int32),
            pltpu.VMEM((N + 16,), jnp.int32),
            pltpu.VMEM((3 * KPAD,), jnp.float32),
        ],
    )(_ballquery_body)
    return fn(inp.reshape(B * 3 * N), idx.reshape(B * S)).reshape(B, S, 3 * KPAD)


def _bn_leaky(h, g, be, maskf, nvalid):
    m = jnp.sum(h * maskf, axis=0, keepdims=True) / nvalid
    v = jnp.sum(((h - m) ** 2) * maskf, axis=0, keepdims=True) / nvalid
    h = (h - m) / jnp.sqrt(v + 1e-5) * g + be
    return jnp.where(h >= 0.0, h, 0.2 * h)


def _conv_body(g_ref, w0, g0, e0, w1, g1, e1, w2, g2, e2, out_ref):
    P = B * S * KPAD
    ii = lax.broadcasted_iota(jnp.int32, (P, 1), 0)
    maskf = ((ii % KPAD) < (KNB + 1)).astype(jnp.float32)
    kmask3 = lax.broadcasted_iota(jnp.int32, (B * S, KPAD, 512), 1) < (KNB + 1)
    nvalid = float(B * S * (KNB + 1))
    h = _dotb(g_ref[...], w0[...])
    h = _bn_leaky(h, g0[...], e0[...], maskf, nvalid)
    h = _bn_leaky(_dotb(h, w1[...]), g1[...], e1[...], maskf, nvalid)
    h = _bn_leaky(_dotb(h, w2[...]), g2[...], e2[...], maskf, nvalid)
    hk = h.reshape(B * S, KPAD, 512)
    hk = jnp.where(kmask3, hk, -jnp.inf)
    out_ref[...] = jnp.max(hk, axis=1)


def _convs(g, p):
    args = [g]
    for j, co in enumerate((64, 256, 512)):
        args += [p["c%d_W" % j].T,
                 p["c%d_g" % j].reshape(1, co),
                 p["c%d_be" % j].reshape(1, co)]
    return pl.pallas_call(
        _conv_body,
        out_shape=jax.ShapeDtypeStruct((B * S, 512), jnp.float32),
    )(*args)


def kernel(hoch_features, inp, params):
    p = params
    x = jnp.transpose(hoch_features, (0, 2, 1))
    for l in range(2):
        wqkv = jnp.concatenate(
            [p["sa%d_Wq" % l], p["sa%d_Wk" % l], p["sa%d_Wv" % l]], axis=1)
        bqkv = jnp.concatenate(
            [p["sa%d_bq" % l], p["sa%d_bk" % l], p["sa%d_bv" % l]]).reshape(1, 3 * C)
        x = _attn_layer(x, wqkv, bqkv, p["sa%d_Wo" % l],
                        p["sa%d_bo" % l].reshape(1, C))
    idx = _fg_argmax(x.reshape(B * N, C), p)
    grouped = _ballquery(inp, idx)
    g = grouped.reshape(B * S, 3, KPAD).transpose(0, 2, 1).reshape(B * S * KPAD, 3)
    out = _convs(g, params)
    return jnp.transpose(out.reshape(B, S, 512), (0, 2, 1))


# speedup vs baseline: 2.8204x; 1.5532x over previous
import functools

import jax
import jax.numpy as jnp
import numpy as np
from jax import lax
from jax.experimental import pallas as pl
from jax.experimental.pallas import tpu as pltpu
from jax.experimental.pallas import tpu_sc as plsc

B, N, C = 8, 2048, 256
H, DH = 4, 64
S = 16
KNB = 32
KPAD = 48
R2 = 0.3 * 0.3


def _dotb(a, b):
    return jnp.dot(a.astype(jnp.bfloat16), b.astype(jnp.bfloat16),
                   preferred_element_type=jnp.float32)


def _attn_body(x_ref, wqkv_ref, bqkv_ref, wo_ref, bo_ref, out_ref):
    x = x_ref[0]
    qkv = _dotb(x, wqkv_ref[...]) + bqkv_ref[...]
    scale = 1.0 / np.sqrt(DH)
    o_cols = []
    for h in range(H):
        q = qkv[:, h * DH:(h + 1) * DH]
        k = qkv[:, C + h * DH:C + (h + 1) * DH]
        v = qkv[:, 2 * C + h * DH:2 * C + (h + 1) * DH]
        logits = lax.dot_general(q.astype(jnp.bfloat16), k.astype(jnp.bfloat16),
                                 (((1,), (1,)), ((), ())),
                                 preferred_element_type=jnp.float32) * scale
        e = jnp.exp(logits)
        s = jnp.sum(e, axis=-1, keepdims=True)
        o_cols.append(_dotb(e, v) / s)
    o = jnp.concatenate(o_cols, axis=-1)
    out_ref[0] = x + _dotb(o, wo_ref[...]) + bo_ref[...]


def _attn_layer(x, wqkv, bqkv, wo, bo):
    return pl.pallas_call(
        _attn_body,
        grid=(B,),
        in_specs=[
            pl.BlockSpec((1, N, C), lambda b: (b, 0, 0)),
            pl.BlockSpec((C, 3 * C), lambda b: (0, 0)),
            pl.BlockSpec((1, 3 * C), lambda b: (0, 0)),
            pl.BlockSpec((C, C), lambda b: (0, 0)),
            pl.BlockSpec((1, C), lambda b: (0, 0)),
        ],
        out_specs=pl.BlockSpec((1, N, C), lambda b: (b, 0, 0)),
        out_shape=jax.ShapeDtypeStruct((B, N, C), jnp.float32),
    )(x, wqkv, bqkv, wo, bo)


def _bn_relu(h, g, be):
    m = jnp.mean(h, axis=0, keepdims=True)
    v = jnp.mean((h - m) ** 2, axis=0, keepdims=True)
    h = (h - m) / jnp.sqrt(v + 1e-5) * g + be
    return jnp.maximum(h, 0.0)


def _fg_body(x_ref, w0, b0, g0, e0, w1, b1, g1, e1, w2, b2, g2, e2, out_ref):
    h = _dotb(x_ref[...], w0[...]) + b0[...]
    h = _bn_relu(h, g0[...], e0[...])
    h = _dotb(h, w1[...]) + b1[...]
    h = _bn_relu(h, g1[...], e1[...])
    h = _dotb(h, w2[...]) + b2[...]
    h = _bn_relu(h, g2[...], e2[...])
    hr = h.reshape(B, N, S)
    mx = jnp.max(hr, axis=1, keepdims=True)
    ii = lax.broadcasted_iota(jnp.int32, (B, N, S), 1)
    out_ref[...] = jnp.min(jnp.where(hr == mx, ii, N), axis=1)


def _fg_argmax(xf, p):
    args = []
    for j, (ci, co) in enumerate(((C, 64), (64, 32), (32, S))):
        args += [p["fg_W%d" % j].T,
                 p["fg_b%d" % j].reshape(1, co),
                 p["fg_g%d" % j].reshape(1, co),
                 p["fg_be%d" % j].reshape(1, co)]
    return pl.pallas_call(
        _fg_body,
        out_shape=jax.ShapeDtypeStruct((B, S), jnp.int32),
    )(xf, *args)


def _ballquery_body(inp_hbm, idx_hbm, out_hbm, xv, yv, zv, idxrow, idxbuf,
                    outbuf):
    info = plsc.get_sparse_core_info()
    nc = info.num_cores
    wid = lax.axis_index("s") * nc + lax.axis_index("c")
    b = wid // 4
    soff = (wid % 4) * 4
    pltpu.sync_copy(inp_hbm.at[pl.ds((b * 3 + 0) * N, N)], xv)
    pltpu.sync_copy(inp_hbm.at[pl.ds((b * 3 + 1) * N, N)], yv)
    pltpu.sync_copy(inp_hbm.at[pl.ds((b * 3 + 2) * N, N)], zv)
    pltpu.sync_copy(idx_hbm.at[pl.ds(b * S, S)], idxrow.at[pl.ds(0, S)])
    lane = lax.iota(jnp.int32, 16)
    zero16 = jnp.zeros((16,), jnp.int32)
    for t in range(4):
        s_loc = soff + t
        qiv = plsc.load_gather(idxrow, [jnp.full((16,), s_loc, jnp.int32)])
        qx = plsc.load_gather(xv, [qiv])
        qy = plsc.load_gather(yv, [qiv])
        qz = plsc.load_gather(zv, [qiv])

        def body(ci, cnt):
            px = xv[pl.ds(ci * 16, 16)]
            py = yv[pl.ds(ci * 16, 16)]
            pz = zv[pl.ds(ci * 16, 16)]
            dx, dy, dz = px - qx, py - qy, pz - qz
            d2 = dx * dx + dy * dy + dz * dz
            msk = d2 <= R2
            vals = lane + ci * 16
            plsc.store_compressed(idxbuf.at[pl.ds(cnt, 16)], vals, mask=msk)
            return cnt + jnp.max(plsc.all_reduce_population_count(msk))

        cnt = lax.fori_loop(0, N // 16, body, jnp.int32(0))
        firstv = plsc.load_gather(idxbuf, [zero16])
        cntv = jnp.full((16,), cnt, jnp.int32)
        sel0 = jnp.where(lane < cntv, idxbuf[pl.ds(0, 16)], firstv)
        sel1 = jnp.where(lane + 16 < cntv, idxbuf[pl.ds(16, 16)], firstv)
        outbuf[pl.ds(0, 16)] = plsc.load_gather(xv, [sel0])
        outbuf[pl.ds(16, 16)] = plsc.load_gather(xv, [sel1])
        outbuf[pl.ds(32, 16)] = qx
        outbuf[pl.ds(48, 16)] = plsc.load_gather(yv, [sel0])
        outbuf[pl.ds(64, 16)] = plsc.load_gather(yv, [sel1])
        outbuf[pl.ds(80, 16)] = qy
        outbuf[pl.ds(96, 16)] = plsc.load_gather(zv, [sel0])
        outbuf[pl.ds(112, 16)] = plsc.load_gather(zv, [sel1])
        outbuf[pl.ds(128, 16)] = qz
        pltpu.sync_copy(outbuf,
                        out_hbm.at[pl.ds((b * S + s_loc) * (3 * KPAD), 3 * KPAD)])


def _ballquery(inp, idx):
    mesh = plsc.VectorSubcoreMesh(core_axis_name="c", subcore_axis_name="s")
    fn = functools.partial(
        pl.kernel,
        mesh=mesh,
        compiler_params=pltpu.CompilerParams(needs_layout_passes=False),
        out_type=jax.ShapeDtypeStruct((B * S * 3 * KPAD,), jnp.float32),
        scratch_types=[
            pltpu.VMEM((N,), jnp.float32),
            pltpu.VMEM((N,), jnp.float32),
            pltpu.VMEM((N,), jnp.float32),
            pltpu.VMEM((128,), jnp.int32),
            pltpu.VMEM((N + 16,), jnp.int32),
            pltpu.VMEM((3 * KPAD,), jnp.float32),
        ],
    )(_ballquery_body)
    return fn(inp.reshape(B * 3 * N), idx.reshape(B * S)).reshape(B, S, 3 * KPAD)


def _bn_leaky(h, g, be, maskf, nvalid):
    m = jnp.sum(h * maskf, axis=0, keepdims=True) / nvalid
    v = jnp.sum(((h - m) ** 2) * maskf, axis=0, keepdims=True) / nvalid
    h = (h - m) / jnp.sqrt(v + 1e-5) * g + be
    return jnp.where(h >= 0.0, h, 0.2 * h)


def _conv_body(g_ref, w0, g0, e0, w1, g1, e1, w2, g2, e2, out_ref):
    P = B * S * KPAD
    ii = lax.broadcasted_iota(jnp.int32, (P, 1), 0)
    maskf = ((ii % KPAD) < (KNB + 1)).astype(jnp.float32)
    kmask3 = lax.broadcasted_iota(jnp.int32, (B * S, KPAD, 512), 1) < (KNB + 1)
    nvalid = float(B * S * (KNB + 1))
    h = _dotb(g_ref[...], w0[...])
    h = _bn_leaky(h, g0[...], e0[...], maskf, nvalid)
    h = _bn_leaky(_dotb(h, w1[...]), g1[...], e1[...], maskf, nvalid)
    h = _bn_leaky(_dotb(h, w2[...]), g2[...], e2[...], maskf, nvalid)
    hk = h.reshape(B * S, KPAD, 512)
    hk = jnp.where(kmask3, hk, -jnp.inf)
    out_ref[...] = jnp.max(hk, axis=1)


def _convs(g, p):
    args = [g]
    for j, co in enumerate((64, 256, 512)):
        args += [p["c%d_W" % j].T,
                 p["c%d_g" % j].reshape(1, co),
                 p["c%d_be" % j].reshape(1, co)]
    return pl.pallas_call(
        _conv_body,
        out_shape=jax.ShapeDtypeStruct((B * S, 512), jnp.float32),
    )(*args)


def kernel(hoch_features, inp, params):
    p = params
    x = jnp.transpose(hoch_features, (0, 2, 1))
    for l in range(2):
        wqkv = jnp.concatenate(
            [p["sa%d_Wq" % l], p["sa%d_Wk" % l], p["sa%d_Wv" % l]], axis=1)
        bqkv = jnp.concatenate(
            [p["sa%d_bq" % l], p["sa%d_bk" % l], p["sa%d_bv" % l]]).reshape(1, 3 * C)
        x = _attn_layer(x, wqkv, bqkv, p["sa%d_Wo" % l],
                        p["sa%d_bo" % l].reshape(1, C))
    idx = _fg_argmax(x.reshape(B * N, C), p)
    grouped = _ballquery(inp, idx)
    g = grouped.reshape(B * S, 3, KPAD).transpose(0, 2, 1).reshape(B * S * KPAD, 3)
    out = _convs(g, params)
    return jnp.transpose(out.reshape(B, S, 512), (0, 2, 1))
